# Initial kernel scaffold; baseline (speedup 1.0000x reference)
import jax
import jax.numpy as jnp
from jax.experimental import pallas as pl


def kernel(node_fea, idx1, idx2, edge_fea, W_full, b_full, gamma1, beta1, gamma2, beta2):
    raise NotImplementedError("write your pallas kernel here")


# speedup vs baseline: 2.7016x; 2.7016x over previous
import functools

import jax
import jax.numpy as jnp
from jax import lax
from jax.experimental import pallas as pl
from jax.experimental.pallas import tpu as pltpu
from jax.experimental.pallas import tpu_sc as plsc

EPS = 1e-5
NC = 2
NS = 16
NW = NC * NS
CH = 128


def _sc_mesh():
    return plsc.VectorSubcoreMesh(
        core_axis_name="c", subcore_axis_name="s", num_cores=NC, num_subcores=NS
    )


def _softplus(x):
    return jnp.maximum(x, 0.0) + jnp.log1p(jnp.exp(-jnp.abs(x)))


def _sigmoid(x):
    return 1.0 / (1.0 + jnp.exp(-x))


def _sc_gather(node_fea, idx1, idx2):
    n, d = node_fea.shape
    e = idx1.shape[0]
    epw = e // NW
    chg = 96
    nch, tail = divmod(epw, chg)
    rpt = (-(-n // NS) + 7) // 8 * 8
    n_pad = NS * rpt

    zcnt = jnp.zeros((n_pad, d), jnp.float32)
    ones = jnp.ones((chg, d), jnp.float32)

    scratch = [
        pltpu.VMEM((chg,), jnp.int32),
        pltpu.VMEM((chg,), jnp.int32),
        pltpu.VMEM((chg, d), jnp.float32),
        pltpu.VMEM((chg, d), jnp.float32),
        pltpu.VMEM((chg, d), jnp.float32),
        pltpu.VMEM_SHARED((n_pad, d), jnp.float32),
        pltpu.SemaphoreType.DMA,
        pltpu.SemaphoreType.DMA,
    ]
    if tail:
        scratch += [
            pltpu.VMEM((tail,), jnp.int32),
            pltpu.VMEM((tail,), jnp.int32),
            pltpu.VMEM((tail, d), jnp.float32),
            pltpu.VMEM((tail, d), jnp.float32),
        ]

    @functools.partial(
        pl.kernel,
        out_type=[
            jax.ShapeDtypeStruct((e, d), jnp.float32),
            jax.ShapeDtypeStruct((e, d), jnp.float32),
            jax.ShapeDtypeStruct((NC, n_pad, d), jnp.float32),
        ],
        mesh=_sc_mesh(),
        scratch_types=scratch,
    )
    def gather_kernel(node_hbm, idx1_hbm, idx2_hbm, zcnt_hbm, ones_hbm,
                      g1_hbm, g2_hbm, pcnt_hbm,
                      i1_v, i2_v, r1_v, r2_v, o_v, scnt, sem1, sem2, *tails):
        cid = lax.axis_index("c")
        sid = lax.axis_index("s")
        wid = sid * NC + cid
        base = wid * epw

        pltpu.sync_copy(zcnt_hbm.at[pl.ds(sid * rpt, rpt)],
                        scnt.at[pl.ds(sid * rpt, rpt)])
        pltpu.sync_copy(ones_hbm, o_v)
        plsc.subcore_barrier()

        def chunk(off, cnt, ia, ib, ra, rb, oo):
            pltpu.sync_copy(idx1_hbm.at[pl.ds(off, cnt)], ia)
            pltpu.sync_copy(idx2_hbm.at[pl.ds(off, cnt)], ib)
            c1 = pltpu.async_copy(node_hbm.at[ia], ra, sem1)
            c2 = pltpu.async_copy(node_hbm.at[ib], rb, sem2)
            pltpu.sync_copy(oo, scnt.at[ia], add=True)
            c1.wait()
            pltpu.sync_copy(ra, g1_hbm.at[pl.ds(off, cnt)])
            c2.wait()
            pltpu.sync_copy(rb, g2_hbm.at[pl.ds(off, cnt)])

        def body(i, carry):
            chunk(base + i * chg, chg, i1_v, i2_v, r1_v, r2_v, o_v)
            return carry

        lax.fori_loop(0, nch, body, 0)
        if tail:
            i1t, i2t, r1t, r2t = tails
            chunk(base + nch * chg, tail, i1t, i2t, r1t, r2t,
                  o_v.at[pl.ds(0, tail)])
        plsc.subcore_barrier()

        pltpu.sync_copy(scnt.at[pl.ds(sid * rpt, rpt)],
                        pcnt_hbm.at[cid, pl.ds(sid * rpt, rpt)])

    return gather_kernel(node_fea, idx1, idx2, zcnt, ones)


def _sc_scatter(msg, idx1, n):
    e, d = msg.shape
    epw = e // NW
    nch, tail = divmod(epw, CH)
    rpt = (-(-n // NS) + 7) // 8 * 8
    n_pad = NS * rpt

    zsum = jnp.zeros((n_pad, d), jnp.float32)

    scratch = [
        pltpu.VMEM((CH,), jnp.int32),
        pltpu.VMEM((CH, d), jnp.float32),
        pltpu.VMEM_SHARED((n_pad, d), jnp.float32),
    ]
    if tail:
        scratch += [
            pltpu.VMEM((tail,), jnp.int32),
            pltpu.VMEM((tail, d), jnp.float32),
        ]

    @functools.partial(
        pl.kernel,
        out_type=[
            jax.ShapeDtypeStruct((NC, n_pad, d), jnp.float32),
        ],
        mesh=_sc_mesh(),
        scratch_types=scratch,
    )
    def scatter_kernel(msg_hbm, idx_hbm, zsum_hbm,
                       psum_hbm, i_v, m_v, ssum, *tails):
        cid = lax.axis_index("c")
        sid = lax.axis_index("s")
        wid = sid * NC + cid
        base = wid * epw

        pltpu.sync_copy(zsum_hbm.at[pl.ds(sid * rpt, rpt)],
                        ssum.at[pl.ds(sid * rpt, rpt)])
        plsc.subcore_barrier()

        def body(i, carry):
            off = base + i * CH
            pltpu.sync_copy(idx_hbm.at[pl.ds(off, CH)], i_v)
            pltpu.sync_copy(msg_hbm.at[pl.ds(off, CH)], m_v)
            pltpu.sync_copy(m_v, ssum.at[i_v], add=True)
            return carry

        lax.fori_loop(0, nch, body, 0)
        if tail:
            it_v, mt_v = tails
            off = base + nch * CH
            pltpu.sync_copy(idx_hbm.at[pl.ds(off, tail)], it_v)
            pltpu.sync_copy(msg_hbm.at[pl.ds(off, tail)], mt_v)
            pltpu.sync_copy(mt_v, ssum.at[it_v], add=True)
        plsc.subcore_barrier()

        pltpu.sync_copy(ssum.at[pl.ds(sid * rpt, rpt)],
                        psum_hbm.at[cid, pl.ds(sid * rpt, rpt)])

    (psum,) = scatter_kernel(msg, idx1, zsum)
    return psum


def _tc_stats(g1, g2, edge, wa, wb, we, b2d, tb):
    e, d = g1.shape
    d2 = b2d.shape[1]
    de = edge.shape[1]
    grid = (e // tb,)

    def body(g1_ref, g2_ref, e_ref, wa_ref, wb_ref, we_ref, b_ref, out_ref):
        i = pl.program_id(0)
        z = (
            jnp.dot(g1_ref[...].astype(jnp.bfloat16), wa_ref[...],
                    preferred_element_type=jnp.float32)
            + jnp.dot(g2_ref[...].astype(jnp.bfloat16), wb_ref[...],
                      preferred_element_type=jnp.float32)
            + jnp.dot(e_ref[...].astype(jnp.bfloat16), we_ref[...],
                      preferred_element_type=jnp.float32)
            + b_ref[...]
        )

        @pl.when(i == 0)
        def _():
            out_ref[...] = jnp.zeros_like(out_ref)

        out_ref[0:1, :] += jnp.sum(z, axis=0, keepdims=True)
        out_ref[1:2, :] += jnp.sum(z * z, axis=0, keepdims=True)

    return pl.pallas_call(
        body,
        grid=grid,
        in_specs=[
            pl.BlockSpec((tb, d), lambda i: (i, 0)),
            pl.BlockSpec((tb, d), lambda i: (i, 0)),
            pl.BlockSpec((tb, de), lambda i: (i, 0)),
            pl.BlockSpec((d, d2), lambda i: (0, 0)),
            pl.BlockSpec((d, d2), lambda i: (0, 0)),
            pl.BlockSpec((de, d2), lambda i: (0, 0)),
            pl.BlockSpec((1, d2), lambda i: (0, 0)),
        ],
        out_specs=pl.BlockSpec((2, d2), lambda i: (0, 0)),
        out_shape=jax.ShapeDtypeStruct((2, d2), jnp.float32),
    )(g1, g2, edge, wa, wb, we, b2d)


def _tc_msg(g1, g2, edge, wa, wb, we, b2d, aff, tb):
    e, d = g1.shape
    d2 = b2d.shape[1]
    de = edge.shape[1]
    grid = (e // tb,)

    def body(g1_ref, g2_ref, e_ref, wa_ref, wb_ref, we_ref, b_ref, a_ref,
             out_ref):
        z = (
            jnp.dot(g1_ref[...].astype(jnp.bfloat16), wa_ref[...],
                    preferred_element_type=jnp.float32)
            + jnp.dot(g2_ref[...].astype(jnp.bfloat16), wb_ref[...],
                      preferred_element_type=jnp.float32)
            + jnp.dot(e_ref[...].astype(jnp.bfloat16), we_ref[...],
                      preferred_element_type=jnp.float32)
            + b_ref[...]
        )
        u = z * a_ref[0:1, :] + a_ref[1:2, :]
        gate = _sigmoid(u[:, :d])
        conv = _softplus(u[:, d:])
        out_ref[...] = gate * conv

    return pl.pallas_call(
        body,
        grid=grid,
        in_specs=[
            pl.BlockSpec((tb, d), lambda i: (i, 0)),
            pl.BlockSpec((tb, d), lambda i: (i, 0)),
            pl.BlockSpec((tb, de), lambda i: (i, 0)),
            pl.BlockSpec((d, d2), lambda i: (0, 0)),
            pl.BlockSpec((d, d2), lambda i: (0, 0)),
            pl.BlockSpec((de, d2), lambda i: (0, 0)),
            pl.BlockSpec((1, d2), lambda i: (0, 0)),
            pl.BlockSpec((2, d2), lambda i: (0, 0)),
        ],
        out_specs=pl.BlockSpec((tb, d), lambda i: (i, 0)),
        out_shape=jax.ShapeDtypeStruct((e, d), jnp.float32),
    )(g1, g2, edge, wa, wb, we, b2d, aff)


def _tc_final(node_fea, psum, pcnt, g2d, b2d, tn):
    n, d = node_fea.shape
    nt = n // tn
    nf = float(n)

    def body(node_ref, s_ref, c_ref, g_ref, b_ref, out_ref, acc1, acc2):
        p = pl.program_id(0)
        t = pl.program_id(1)
        s = s_ref[0] + s_ref[1]
        cnt = c_ref[0, :, 0:1] + c_ref[1, :, 0:1]
        agg = s / jnp.maximum(cnt, 1.0)

        @pl.when((p == 0) & (t == 0))
        def _():
            acc1[...] = jnp.zeros_like(acc1)
            acc2[...] = jnp.zeros_like(acc2)

        @pl.when(p == 0)
        def _():
            acc1[...] += jnp.sum(agg, axis=0, keepdims=True)
            acc2[...] += jnp.sum(agg * agg, axis=0, keepdims=True)
            out_ref[...] = agg

        @pl.when(p == 1)
        def _():
            mu = acc1[...] / nf
            var = acc2[...] / nf - mu * mu
            inv = 1.0 / jnp.sqrt(var + EPS)
            bn = (agg - mu) * inv * g_ref[...] + b_ref[...]
            out_ref[...] = _softplus(node_ref[...] + bn)

    return pl.pallas_call(
        body,
        grid=(2, nt),
        in_specs=[
            pl.BlockSpec((tn, d), lambda p, t: (t, 0)),
            pl.BlockSpec((NC, tn, d), lambda p, t: (0, t, 0)),
            pl.BlockSpec((NC, tn, d), lambda p, t: (0, t, 0)),
            pl.BlockSpec((1, d), lambda p, t: (0, 0)),
            pl.BlockSpec((1, d), lambda p, t: (0, 0)),
        ],
        out_specs=pl.BlockSpec((tn, d), lambda p, t: (t, 0)),
        out_shape=jax.ShapeDtypeStruct((n, d), jnp.float32),
        scratch_shapes=[
            pltpu.VMEM((1, d), jnp.float32),
            pltpu.VMEM((1, d), jnp.float32),
        ],
    )(node_fea, psum, pcnt, g2d, b2d)


def kernel(node_fea, idx1, idx2, edge_fea, W_full, b_full, gamma1, beta1,
           gamma2, beta2):
    n, d = node_fea.shape
    e = idx1.shape[0]
    d2 = 2 * d

    wa = W_full[:, :d].T.astype(jnp.bfloat16)
    wb = W_full[:, d:2 * d].T.astype(jnp.bfloat16)
    we = W_full[:, 2 * d:].T.astype(jnp.bfloat16)
    b2d = b_full.reshape(1, d2)

    g1, g2, pcnt = _sc_gather(node_fea, idx1, idx2)

    s12 = _tc_stats(g1, g2, edge_fea, wa, wb, we, b2d, tb=1280)
    mean = s12[0] / e
    var = s12[1] / e - mean * mean
    a = gamma1 / jnp.sqrt(var + EPS)
    c = beta1 - mean * a
    aff = jnp.stack([a, c])

    msg = _tc_msg(g1, g2, edge_fea, wa, wb, we, b2d, aff, tb=1280)

    psum = _sc_scatter(msg, idx1, n)

    out = _tc_final(node_fea, psum, pcnt, gamma2.reshape(1, d),
                    beta2.reshape(1, d), tn=2000)
    return out


# speedup vs baseline: 2.8508x; 1.0552x over previous
import functools

import jax
import jax.numpy as jnp
from jax import lax
from jax.experimental import pallas as pl
from jax.experimental.pallas import tpu as pltpu
from jax.experimental.pallas import tpu_sc as plsc

EPS = 1e-5
NC = 2
NS = 16
NW = NC * NS
CH = 128


def _sc_mesh():
    return plsc.VectorSubcoreMesh(
        core_axis_name="c", subcore_axis_name="s", num_cores=NC, num_subcores=NS
    )


def _softplus(x):
    return jnp.maximum(x, 0.0) + jnp.log1p(jnp.exp(-jnp.abs(x)))


def _sigmoid(x):
    return 1.0 / (1.0 + jnp.exp(-x))


def _sc_gather(node_fea, idx1, idx2):
    n, d = node_fea.shape
    e = idx1.shape[0]
    epw = e // NW
    chg = 96
    nch, tail = divmod(epw, chg)
    rpt = (-(-n // NS) + 7) // 8 * 8
    n_pad = NS * rpt

    zcnt = jnp.zeros((n_pad, d), jnp.float32)
    ones = jnp.ones((chg, d), jnp.float32)

    scratch = [
        pltpu.VMEM((chg,), jnp.int32),
        pltpu.VMEM((chg,), jnp.int32),
        pltpu.VMEM((chg, d), jnp.float32),
        pltpu.VMEM((chg, d), jnp.float32),
        pltpu.VMEM((chg, d), jnp.float32),
        pltpu.VMEM_SHARED((n_pad, d), jnp.float32),
        pltpu.SemaphoreType.DMA,
        pltpu.SemaphoreType.DMA,
    ]
    if tail:
        scratch += [
            pltpu.VMEM((tail,), jnp.int32),
            pltpu.VMEM((tail,), jnp.int32),
            pltpu.VMEM((tail, d), jnp.float32),
            pltpu.VMEM((tail, d), jnp.float32),
        ]

    @functools.partial(
        pl.kernel,
        out_type=[
            jax.ShapeDtypeStruct((e, d), jnp.float32),
            jax.ShapeDtypeStruct((e, d), jnp.float32),
            jax.ShapeDtypeStruct((NC, n_pad, d), jnp.float32),
        ],
        mesh=_sc_mesh(),
        scratch_types=scratch,
    )
    def gather_kernel(node_hbm, idx1_hbm, idx2_hbm, zcnt_hbm, ones_hbm,
                      g1_hbm, g2_hbm, pcnt_hbm,
                      i1_v, i2_v, r1_v, r2_v, o_v, scnt, sem1, sem2, *tails):
        cid = lax.axis_index("c")
        sid = lax.axis_index("s")
        wid = sid * NC + cid
        base = wid * epw

        pltpu.sync_copy(zcnt_hbm.at[pl.ds(sid * rpt, rpt)],
                        scnt.at[pl.ds(sid * rpt, rpt)])
        pltpu.sync_copy(ones_hbm, o_v)
        plsc.subcore_barrier()

        def chunk(off, cnt, ia, ib, ra, rb, oo):
            pltpu.sync_copy(idx1_hbm.at[pl.ds(off, cnt)], ia)
            pltpu.sync_copy(idx2_hbm.at[pl.ds(off, cnt)], ib)
            c1 = pltpu.async_copy(node_hbm.at[ia], ra, sem1)
            c2 = pltpu.async_copy(node_hbm.at[ib], rb, sem2)
            pltpu.sync_copy(oo, scnt.at[ia], add=True)
            c1.wait()
            pltpu.sync_copy(ra, g1_hbm.at[pl.ds(off, cnt)])
            c2.wait()
            pltpu.sync_copy(rb, g2_hbm.at[pl.ds(off, cnt)])

        def body(i, carry):
            chunk(base + i * chg, chg, i1_v, i2_v, r1_v, r2_v, o_v)
            return carry

        lax.fori_loop(0, nch, body, 0)
        if tail:
            i1t, i2t, r1t, r2t = tails
            chunk(base + nch * chg, tail, i1t, i2t, r1t, r2t,
                  o_v.at[pl.ds(0, tail)])
        plsc.subcore_barrier()

        pltpu.sync_copy(scnt.at[pl.ds(sid * rpt, rpt)],
                        pcnt_hbm.at[cid, pl.ds(sid * rpt, rpt)])

    return gather_kernel(node_fea, idx1, idx2, zcnt, ones)


def _sc_scatter(msg, idx1, n):
    e, d = msg.shape
    epw = e // NW
    nch, tail = divmod(epw, CH)
    rpt = (-(-n // NS) + 7) // 8 * 8
    n_pad = NS * rpt

    zsum = jnp.zeros((n_pad, d), jnp.float32)

    scratch = [
        pltpu.VMEM((CH,), jnp.int32),
        pltpu.VMEM((CH, d), jnp.float32),
        pltpu.VMEM_SHARED((n_pad, d), jnp.float32),
    ]
    if tail:
        scratch += [
            pltpu.VMEM((tail,), jnp.int32),
            pltpu.VMEM((tail, d), jnp.float32),
        ]

    @functools.partial(
        pl.kernel,
        out_type=[
            jax.ShapeDtypeStruct((NC, n_pad, d), jnp.float32),
        ],
        mesh=_sc_mesh(),
        scratch_types=scratch,
    )
    def scatter_kernel(msg_hbm, idx_hbm, zsum_hbm,
                       psum_hbm, i_v, m_v, ssum, *tails):
        cid = lax.axis_index("c")
        sid = lax.axis_index("s")
        wid = sid * NC + cid
        base = wid * epw

        pltpu.sync_copy(zsum_hbm.at[pl.ds(sid * rpt, rpt)],
                        ssum.at[pl.ds(sid * rpt, rpt)])
        plsc.subcore_barrier()

        def body(i, carry):
            off = base + i * CH
            pltpu.sync_copy(idx_hbm.at[pl.ds(off, CH)], i_v)
            pltpu.sync_copy(msg_hbm.at[pl.ds(off, CH)], m_v)
            pltpu.sync_copy(m_v, ssum.at[i_v], add=True)
            return carry

        lax.fori_loop(0, nch, body, 0)
        if tail:
            it_v, mt_v = tails
            off = base + nch * CH
            pltpu.sync_copy(idx_hbm.at[pl.ds(off, tail)], it_v)
            pltpu.sync_copy(msg_hbm.at[pl.ds(off, tail)], mt_v)
            pltpu.sync_copy(mt_v, ssum.at[it_v], add=True)
        plsc.subcore_barrier()

        pltpu.sync_copy(ssum.at[pl.ds(sid * rpt, rpt)],
                        psum_hbm.at[cid, pl.ds(sid * rpt, rpt)])

    (psum,) = scatter_kernel(msg, idx1, zsum)
    return psum


def _tc_stats(g1, g2, edge, wa, wb, we, b2d, tb):
    e, d = g1.shape
    d2 = b2d.shape[1]
    de = edge.shape[1]
    grid = (e // tb,)

    def body(g1_ref, g2_ref, e_ref, wa_ref, wb_ref, we_ref, b_ref,
             out_ref, z_ref):
        i = pl.program_id(0)
        z = (
            jnp.dot(g1_ref[...].astype(jnp.bfloat16), wa_ref[...],
                    preferred_element_type=jnp.float32)
            + jnp.dot(g2_ref[...].astype(jnp.bfloat16), wb_ref[...],
                      preferred_element_type=jnp.float32)
            + jnp.dot(e_ref[...].astype(jnp.bfloat16), we_ref[...],
                      preferred_element_type=jnp.float32)
            + b_ref[...]
        )
        z_ref[...] = z.astype(jnp.bfloat16)

        @pl.when(i == 0)
        def _():
            out_ref[...] = jnp.zeros_like(out_ref)

        out_ref[0:1, :] += jnp.sum(z, axis=0, keepdims=True)
        out_ref[1:2, :] += jnp.sum(z * z, axis=0, keepdims=True)

    return pl.pallas_call(
        body,
        grid=grid,
        in_specs=[
            pl.BlockSpec((tb, d), lambda i: (i, 0)),
            pl.BlockSpec((tb, d), lambda i: (i, 0)),
            pl.BlockSpec((tb, de), lambda i: (i, 0)),
            pl.BlockSpec((d, d2), lambda i: (0, 0)),
            pl.BlockSpec((d, d2), lambda i: (0, 0)),
            pl.BlockSpec((de, d2), lambda i: (0, 0)),
            pl.BlockSpec((1, d2), lambda i: (0, 0)),
        ],
        out_specs=[
            pl.BlockSpec((2, d2), lambda i: (0, 0)),
            pl.BlockSpec((tb, d2), lambda i: (i, 0)),
        ],
        out_shape=[
            jax.ShapeDtypeStruct((2, d2), jnp.float32),
            jax.ShapeDtypeStruct((e, d2), jnp.bfloat16),
        ],
    )(g1, g2, edge, wa, wb, we, b2d)


def _tc_msg(z, aff, d, tb):
    e, d2 = z.shape
    grid = (e // tb,)

    def body(z_ref, a_ref, out_ref):
        u = z_ref[...].astype(jnp.float32) * a_ref[0:1, :] + a_ref[1:2, :]
        gate = _sigmoid(u[:, :d])
        conv = _softplus(u[:, d:])
        out_ref[...] = gate * conv

    return pl.pallas_call(
        body,
        grid=grid,
        in_specs=[
            pl.BlockSpec((tb, d2), lambda i: (i, 0)),
            pl.BlockSpec((2, d2), lambda i: (0, 0)),
        ],
        out_specs=pl.BlockSpec((tb, d), lambda i: (i, 0)),
        out_shape=jax.ShapeDtypeStruct((e, d), jnp.float32),
    )(z, aff)


def _tc_final(node_fea, psum, pcnt, g2d, b2d, tn):
    n, d = node_fea.shape
    nt = n // tn
    nf = float(n)

    def body(node_ref, s_ref, c_ref, g_ref, b_ref, out_ref, acc1, acc2):
        p = pl.program_id(0)
        t = pl.program_id(1)
        s = s_ref[0] + s_ref[1]
        cnt = c_ref[0, :, 0:1] + c_ref[1, :, 0:1]
        agg = s / jnp.maximum(cnt, 1.0)

        @pl.when((p == 0) & (t == 0))
        def _():
            acc1[...] = jnp.zeros_like(acc1)
            acc2[...] = jnp.zeros_like(acc2)

        @pl.when(p == 0)
        def _():
            acc1[...] += jnp.sum(agg, axis=0, keepdims=True)
            acc2[...] += jnp.sum(agg * agg, axis=0, keepdims=True)
            out_ref[...] = agg

        @pl.when(p == 1)
        def _():
            mu = acc1[...] / nf
            var = acc2[...] / nf - mu * mu
            inv = 1.0 / jnp.sqrt(var + EPS)
            bn = (agg - mu) * inv * g_ref[...] + b_ref[...]
            out_ref[...] = _softplus(node_ref[...] + bn)

    return pl.pallas_call(
        body,
        grid=(2, nt),
        in_specs=[
            pl.BlockSpec((tn, d), lambda p, t: (t, 0)),
            pl.BlockSpec((NC, tn, d), lambda p, t: (0, t, 0)),
            pl.BlockSpec((NC, tn, d), lambda p, t: (0, t, 0)),
            pl.BlockSpec((1, d), lambda p, t: (0, 0)),
            pl.BlockSpec((1, d), lambda p, t: (0, 0)),
        ],
        out_specs=pl.BlockSpec((tn, d), lambda p, t: (t, 0)),
        out_shape=jax.ShapeDtypeStruct((n, d), jnp.float32),
        scratch_shapes=[
            pltpu.VMEM((1, d), jnp.float32),
            pltpu.VMEM((1, d), jnp.float32),
        ],
    )(node_fea, psum, pcnt, g2d, b2d)


def kernel(node_fea, idx1, idx2, edge_fea, W_full, b_full, gamma1, beta1,
           gamma2, beta2):
    n, d = node_fea.shape
    e = idx1.shape[0]
    d2 = 2 * d

    wa = W_full[:, :d].T.astype(jnp.bfloat16)
    wb = W_full[:, d:2 * d].T.astype(jnp.bfloat16)
    we = W_full[:, 2 * d:].T.astype(jnp.bfloat16)
    b2d = b_full.reshape(1, d2)

    g1, g2, pcnt = _sc_gather(node_fea, idx1, idx2)

    s12, zst = _tc_stats(g1, g2, edge_fea, wa, wb, we, b2d, tb=1280)
    mean = s12[0] / e
    var = s12[1] / e - mean * mean
    a = gamma1 / jnp.sqrt(var + EPS)
    c = beta1 - mean * a
    aff = jnp.stack([a, c])

    msg = _tc_msg(zst, aff, d, tb=1280)

    psum = _sc_scatter(msg, idx1, n)

    out = _tc_final(node_fea, psum, pcnt, gamma2.reshape(1, d),
                    beta2.reshape(1, d), tn=2000)
    return out


# speedup vs baseline: 3.2615x; 1.1441x over previous
import functools

import jax
import jax.numpy as jnp
from jax import lax
from jax.experimental import pallas as pl
from jax.experimental.pallas import tpu as pltpu
from jax.experimental.pallas import tpu_sc as plsc

EPS = 1e-5
NC = 2
NS = 16
NW = NC * NS
CH = 128


def _sc_mesh():
    return plsc.VectorSubcoreMesh(
        core_axis_name="c", subcore_axis_name="s", num_cores=NC, num_subcores=NS
    )


def _softplus(x):
    return jnp.maximum(x, 0.0) + jnp.log1p(jnp.exp(-jnp.abs(x)))


def _sigmoid(x):
    return 1.0 / (1.0 + jnp.exp(-x))


def _sc_gather(node_fea, idx1, idx2, cnt_init):
    n, d = node_fea.shape
    e = idx1.shape[0]
    epw = e // NW
    chg = 96
    nch, tail = divmod(epw, chg)
    rpt = (-(-n // NS) + 7) // 8 * 8
    n_pad = NS * rpt

    ones = jnp.ones((chg, d), jnp.float32)

    scratch = [
        pltpu.VMEM((chg,), jnp.int32),
        pltpu.VMEM((chg,), jnp.int32),
        pltpu.VMEM((chg, d), jnp.float32),
        pltpu.VMEM((chg, d), jnp.float32),
        pltpu.VMEM((chg, d), jnp.float32),
        pltpu.VMEM_SHARED((n_pad, d), jnp.float32),
        pltpu.SemaphoreType.DMA,
        pltpu.SemaphoreType.DMA,
    ]
    if tail:
        scratch += [
            pltpu.VMEM((tail,), jnp.int32),
            pltpu.VMEM((tail,), jnp.int32),
            pltpu.VMEM((tail, d), jnp.float32),
            pltpu.VMEM((tail, d), jnp.float32),
        ]

    @functools.partial(
        pl.kernel,
        out_type=[
            jax.ShapeDtypeStruct((e, d), jnp.float32),
            jax.ShapeDtypeStruct((e, d), jnp.float32),
            jax.ShapeDtypeStruct((NC, n_pad, d), jnp.float32),
        ],
        mesh=_sc_mesh(),
        scratch_types=scratch,
    )
    def gather_kernel(node_hbm, idx1_hbm, idx2_hbm, cinit_hbm, ones_hbm,
                      g1_hbm, g2_hbm, pcnt_hbm,
                      i1_v, i2_v, r1_v, r2_v, o_v, scnt, sem1, sem2, *tails):
        cid = lax.axis_index("c")
        sid = lax.axis_index("s")
        wid = sid * NC + cid
        base = wid * epw

        pltpu.sync_copy(cinit_hbm.at[cid, pl.ds(sid * rpt, rpt)],
                        scnt.at[pl.ds(sid * rpt, rpt)])
        pltpu.sync_copy(ones_hbm, o_v)
        plsc.subcore_barrier()

        def chunk(off, cnt, ia, ib, ra, rb, oo):
            pltpu.sync_copy(idx1_hbm.at[pl.ds(off, cnt)], ia)
            pltpu.sync_copy(idx2_hbm.at[pl.ds(off, cnt)], ib)
            c1 = pltpu.async_copy(node_hbm.at[ia], ra, sem1)
            c2 = pltpu.async_copy(node_hbm.at[ib], rb, sem2)
            pltpu.sync_copy(oo, scnt.at[ia], add=True)
            c1.wait()
            pltpu.sync_copy(ra, g1_hbm.at[pl.ds(off, cnt)])
            c2.wait()
            pltpu.sync_copy(rb, g2_hbm.at[pl.ds(off, cnt)])

        def body(i, carry):
            chunk(base + i * chg, chg, i1_v, i2_v, r1_v, r2_v, o_v)
            return carry

        lax.fori_loop(0, nch, body, 0)
        if tail:
            i1t, i2t, r1t, r2t = tails
            chunk(base + nch * chg, tail, i1t, i2t, r1t, r2t,
                  o_v.at[pl.ds(0, tail)])
        plsc.subcore_barrier()

        pltpu.sync_copy(scnt.at[pl.ds(sid * rpt, rpt)],
                        pcnt_hbm.at[cid, pl.ds(sid * rpt, rpt)])

    return gather_kernel(node_fea, idx1, idx2, cnt_init, ones)


def _sc_scatter(msg, idx1, n, sum_init):
    e, d = msg.shape
    epw = e // NW
    nch, tail = divmod(epw, CH)
    rpt = (-(-n // NS) + 7) // 8 * 8
    n_pad = NS * rpt

    scratch = [
        pltpu.VMEM((CH,), jnp.int32),
        pltpu.VMEM((CH, d), jnp.float32),
        pltpu.VMEM_SHARED((n_pad, d), jnp.float32),
    ]
    if tail:
        scratch += [
            pltpu.VMEM((tail,), jnp.int32),
            pltpu.VMEM((tail, d), jnp.float32),
        ]

    @functools.partial(
        pl.kernel,
        out_type=[
            jax.ShapeDtypeStruct((NC, n_pad, d), jnp.float32),
        ],
        mesh=_sc_mesh(),
        scratch_types=scratch,
    )
    def scatter_kernel(msg_hbm, idx_hbm, sinit_hbm,
                       psum_hbm, i_v, m_v, ssum, *tails):
        cid = lax.axis_index("c")
        sid = lax.axis_index("s")
        wid = sid * NC + cid
        base = wid * epw

        pltpu.sync_copy(sinit_hbm.at[cid, pl.ds(sid * rpt, rpt)],
                        ssum.at[pl.ds(sid * rpt, rpt)])
        plsc.subcore_barrier()

        def body(i, carry):
            off = base + i * CH
            pltpu.sync_copy(idx_hbm.at[pl.ds(off, CH)], i_v)
            pltpu.sync_copy(msg_hbm.at[pl.ds(off, CH)], m_v)
            pltpu.sync_copy(m_v, ssum.at[i_v], add=True)
            return carry

        lax.fori_loop(0, nch, body, 0)
        if tail:
            it_v, mt_v = tails
            off = base + nch * CH
            pltpu.sync_copy(idx_hbm.at[pl.ds(off, tail)], it_v)
            pltpu.sync_copy(msg_hbm.at[pl.ds(off, tail)], mt_v)
            pltpu.sync_copy(mt_v, ssum.at[it_v], add=True)
        plsc.subcore_barrier()

        pltpu.sync_copy(ssum.at[pl.ds(sid * rpt, rpt)],
                        psum_hbm.at[cid, pl.ds(sid * rpt, rpt)])

    (psum,) = scatter_kernel(msg, idx1, sum_init)
    return psum


def _tc_stats(g1, g2, edge, wa, wb, we, b2d, tb):
    e, d = g1.shape
    d2 = b2d.shape[1]
    de = edge.shape[1]
    grid = (e // tb,)

    def body(g1_ref, g2_ref, e_ref, wa_ref, wb_ref, we_ref, b_ref,
             out_ref, z_ref):
        i = pl.program_id(0)
        z = (
            jnp.dot(g1_ref[...].astype(jnp.bfloat16), wa_ref[...],
                    preferred_element_type=jnp.float32)
            + jnp.dot(g2_ref[...].astype(jnp.bfloat16), wb_ref[...],
                      preferred_element_type=jnp.float32)
            + jnp.dot(e_ref[...].astype(jnp.bfloat16), we_ref[...],
                      preferred_element_type=jnp.float32)
            + b_ref[...]
        )
        z_ref[...] = z.astype(jnp.bfloat16)

        @pl.when(i == 0)
        def _():
            out_ref[...] = jnp.zeros_like(out_ref)

        out_ref[0:1, :] += jnp.sum(z, axis=0, keepdims=True)
        out_ref[1:2, :] += jnp.sum(z * z, axis=0, keepdims=True)

    return pl.pallas_call(
        body,
        grid=grid,
        in_specs=[
            pl.BlockSpec((tb, d), lambda i: (i, 0)),
            pl.BlockSpec((tb, d), lambda i: (i, 0)),
            pl.BlockSpec((tb, de), lambda i: (i, 0)),
            pl.BlockSpec((d, d2), lambda i: (0, 0)),
            pl.BlockSpec((d, d2), lambda i: (0, 0)),
            pl.BlockSpec((de, d2), lambda i: (0, 0)),
            pl.BlockSpec((1, d2), lambda i: (0, 0)),
        ],
        out_specs=[
            pl.BlockSpec((2, d2), lambda i: (0, 0)),
            pl.BlockSpec((tb, d2), lambda i: (i, 0)),
        ],
        out_shape=[
            jax.ShapeDtypeStruct((2, d2), jnp.float32),
            jax.ShapeDtypeStruct((e, d2), jnp.bfloat16),
        ],
    )(g1, g2, edge, wa, wb, we, b2d)


def _tc_msg(z, aff, d, tb):
    e, d2 = z.shape
    grid = (e // tb,)

    def body(z_ref, a_ref, out_ref):
        u = z_ref[...].astype(jnp.float32) * a_ref[0:1, :] + a_ref[1:2, :]
        gate = _sigmoid(u[:, :d])
        conv = _softplus(u[:, d:])
        out_ref[...] = gate * conv

    return pl.pallas_call(
        body,
        grid=grid,
        in_specs=[
            pl.BlockSpec((tb, d2), lambda i: (i, 0)),
            pl.BlockSpec((2, d2), lambda i: (0, 0)),
        ],
        out_specs=pl.BlockSpec((tb, d), lambda i: (i, 0)),
        out_shape=jax.ShapeDtypeStruct((e, d), jnp.float32),
    )(z, aff)


def _tc_final(node_fea, psum, pcnt, g2d, b2d, tn):
    n, d = node_fea.shape
    nt = n // tn
    nf = float(n)

    def body(node_ref, s_ref, c_ref, g_ref, b_ref, out_ref, acc1, acc2):
        p = pl.program_id(0)
        t = pl.program_id(1)
        s = s_ref[0] + s_ref[1]
        cnt = c_ref[0, :, 0:1] + c_ref[1, :, 0:1]
        agg = s / jnp.maximum(cnt, 1.0)

        @pl.when((p == 0) & (t == 0))
        def _():
            acc1[...] = jnp.zeros_like(acc1)
            acc2[...] = jnp.zeros_like(acc2)

        @pl.when(p == 0)
        def _():
            acc1[...] += jnp.sum(agg, axis=0, keepdims=True)
            acc2[...] += jnp.sum(agg * agg, axis=0, keepdims=True)
            out_ref[...] = agg

        @pl.when(p == 1)
        def _():
            mu = acc1[...] / nf
            var = acc2[...] / nf - mu * mu
            inv = 1.0 / jnp.sqrt(var + EPS)
            bn = (agg - mu) * inv * g_ref[...] + b_ref[...]
            out_ref[...] = _softplus(node_ref[...] + bn)

    return pl.pallas_call(
        body,
        grid=(2, nt),
        in_specs=[
            pl.BlockSpec((tn, d), lambda p, t: (t, 0)),
            pl.BlockSpec((NC, tn, d), lambda p, t: (0, t, 0)),
            pl.BlockSpec((NC, tn, d), lambda p, t: (0, t, 0)),
            pl.BlockSpec((1, d), lambda p, t: (0, 0)),
            pl.BlockSpec((1, d), lambda p, t: (0, 0)),
        ],
        out_specs=pl.BlockSpec((tn, d), lambda p, t: (t, 0)),
        out_shape=jax.ShapeDtypeStruct((n, d), jnp.float32),
        scratch_shapes=[
            pltpu.VMEM((1, d), jnp.float32),
            pltpu.VMEM((1, d), jnp.float32),
        ],
    )(node_fea, psum, pcnt, g2d, b2d)


def kernel(node_fea, idx1, idx2, edge_fea, W_full, b_full, gamma1, beta1,
           gamma2, beta2):
    n, d = node_fea.shape
    e = idx1.shape[0]
    d2 = 2 * d
    rpt = (-(-n // NS) + 7) // 8 * 8
    n_pad = NS * rpt

    wa = W_full[:, :d].T.astype(jnp.bfloat16)
    wb = W_full[:, d:2 * d].T.astype(jnp.bfloat16)
    we = W_full[:, 2 * d:].T.astype(jnp.bfloat16)
    b2d = b_full.reshape(1, d2)
    zpart = jnp.zeros((NC, n_pad, d), jnp.float32)

    h = e // 2
    i1a, i1b = idx1[:h], idx1[h:]
    i2a, i2b = idx2[:h], idx2[h:]
    ea, eb = edge_fea[:h], edge_fea[h:]

    g1a, g2a, pca = _sc_gather(node_fea, i1a, i2a, zpart)
    g1b, g2b, pcnt = _sc_gather(node_fea, i1b, i2b, pca)

    s12a, za = _tc_stats(g1a, g2a, ea, wa, wb, we, b2d, tb=1600)
    s12b, zb = _tc_stats(g1b, g2b, eb, wa, wb, we, b2d, tb=1600)
    s12 = s12a + s12b
    mean = s12[0] / e
    var = s12[1] / e - mean * mean
    a = gamma1 / jnp.sqrt(var + EPS)
    c = beta1 - mean * a
    aff = jnp.stack([a, c])

    ma = _tc_msg(za, aff, d, tb=1600)
    mb = _tc_msg(zb, aff, d, tb=1600)

    psa = _sc_scatter(ma, i1a, n, zpart)
    psum = _sc_scatter(mb, i1b, n, psa)

    out = _tc_final(node_fea, psum, pcnt, gamma2.reshape(1, d),
                    beta2.reshape(1, d), tn=2000)
    return out


# speedup vs baseline: 3.3253x; 1.0196x over previous
import functools

import jax
import jax.numpy as jnp
from jax import lax
from jax.experimental import pallas as pl
from jax.experimental.pallas import tpu as pltpu
from jax.experimental.pallas import tpu_sc as plsc

EPS = 1e-5
NC = 2
NS = 16
NW = NC * NS
CH = 128


def _sc_mesh():
    return plsc.VectorSubcoreMesh(
        core_axis_name="c", subcore_axis_name="s", num_cores=NC, num_subcores=NS
    )


def _softplus(x):
    return jnp.maximum(x, 0.0) + jnp.log1p(jnp.exp(-jnp.abs(x)))


def _sigmoid(x):
    return 1.0 / (1.0 + jnp.exp(-x))


def _sc_gather(node_fea, idx1, idx2, cnt_init, eoff, esz):
    n, d = node_fea.shape
    e = esz
    epw = e // NW
    chg = 96
    nch, tail = divmod(epw, chg)
    rpt = (-(-n // NS) + 7) // 8 * 8
    n_pad = NS * rpt

    ones = jnp.ones((chg, d), jnp.float32)

    scratch = [
        pltpu.VMEM((chg,), jnp.int32),
        pltpu.VMEM((chg,), jnp.int32),
        pltpu.VMEM((chg, d), jnp.float32),
        pltpu.VMEM((chg, d), jnp.float32),
        pltpu.VMEM((chg, d), jnp.float32),
        pltpu.VMEM_SHARED((n_pad, d), jnp.float32),
        pltpu.SemaphoreType.DMA,
        pltpu.SemaphoreType.DMA,
    ]
    if tail:
        scratch += [
            pltpu.VMEM((tail,), jnp.int32),
            pltpu.VMEM((tail,), jnp.int32),
            pltpu.VMEM((tail, d), jnp.float32),
            pltpu.VMEM((tail, d), jnp.float32),
        ]

    @functools.partial(
        pl.kernel,
        out_type=[
            jax.ShapeDtypeStruct((e, d), jnp.float32),
            jax.ShapeDtypeStruct((e, d), jnp.float32),
            jax.ShapeDtypeStruct((NC, n_pad, d), jnp.float32),
        ],
        mesh=_sc_mesh(),
        scratch_types=scratch,
    )
    def gather_kernel(node_hbm, idx1_hbm, idx2_hbm, cinit_hbm, ones_hbm,
                      g1_hbm, g2_hbm, pcnt_hbm,
                      i1_v, i2_v, r1_v, r2_v, o_v, scnt, sem1, sem2, *tails):
        cid = lax.axis_index("c")
        sid = lax.axis_index("s")
        wid = sid * NC + cid
        base = wid * epw

        pltpu.sync_copy(cinit_hbm.at[cid, pl.ds(sid * rpt, rpt)],
                        scnt.at[pl.ds(sid * rpt, rpt)])
        pltpu.sync_copy(ones_hbm, o_v)
        plsc.subcore_barrier()

        def chunk(off, cnt, ia, ib, ra, rb, oo):
            pltpu.sync_copy(idx1_hbm.at[pl.ds(eoff + off, cnt)], ia)
            pltpu.sync_copy(idx2_hbm.at[pl.ds(eoff + off, cnt)], ib)
            c1 = pltpu.async_copy(node_hbm.at[ia], ra, sem1)
            c2 = pltpu.async_copy(node_hbm.at[ib], rb, sem2)
            pltpu.sync_copy(oo, scnt.at[ia], add=True)
            c1.wait()
            pltpu.sync_copy(ra, g1_hbm.at[pl.ds(off, cnt)])
            c2.wait()
            pltpu.sync_copy(rb, g2_hbm.at[pl.ds(off, cnt)])

        def body(i, carry):
            chunk(base + i * chg, chg, i1_v, i2_v, r1_v, r2_v, o_v)
            return carry

        lax.fori_loop(0, nch, body, 0)
        if tail:
            i1t, i2t, r1t, r2t = tails
            chunk(base + nch * chg, tail, i1t, i2t, r1t, r2t,
                  o_v.at[pl.ds(0, tail)])
        plsc.subcore_barrier()

        pltpu.sync_copy(scnt.at[pl.ds(sid * rpt, rpt)],
                        pcnt_hbm.at[cid, pl.ds(sid * rpt, rpt)])

    return gather_kernel(node_fea, idx1, idx2, cnt_init, ones)


def _sc_scatter(msg, idx1, n, sum_init, eoff):
    e, d = msg.shape
    epw = e // NW
    nch, tail = divmod(epw, CH)
    rpt = (-(-n // NS) + 7) // 8 * 8
    n_pad = NS * rpt

    scratch = [
        pltpu.VMEM((CH,), jnp.int32),
        pltpu.VMEM((CH, d), jnp.float32),
        pltpu.VMEM_SHARED((n_pad, d), jnp.float32),
    ]
    if tail:
        scratch += [
            pltpu.VMEM((tail,), jnp.int32),
            pltpu.VMEM((tail, d), jnp.float32),
        ]

    @functools.partial(
        pl.kernel,
        out_type=[
            jax.ShapeDtypeStruct((NC, n_pad, d), jnp.float32),
        ],
        mesh=_sc_mesh(),
        scratch_types=scratch,
    )
    def scatter_kernel(msg_hbm, idx_hbm, sinit_hbm,
                       psum_hbm, i_v, m_v, ssum, *tails):
        cid = lax.axis_index("c")
        sid = lax.axis_index("s")
        wid = sid * NC + cid
        base = wid * epw

        pltpu.sync_copy(sinit_hbm.at[cid, pl.ds(sid * rpt, rpt)],
                        ssum.at[pl.ds(sid * rpt, rpt)])
        plsc.subcore_barrier()

        def body(i, carry):
            off = base + i * CH
            pltpu.sync_copy(idx_hbm.at[pl.ds(eoff + off, CH)], i_v)
            pltpu.sync_copy(msg_hbm.at[pl.ds(off, CH)], m_v)
            pltpu.sync_copy(m_v, ssum.at[i_v], add=True)
            return carry

        lax.fori_loop(0, nch, body, 0)
        if tail:
            it_v, mt_v = tails
            off = base + nch * CH
            pltpu.sync_copy(idx_hbm.at[pl.ds(eoff + off, tail)], it_v)
            pltpu.sync_copy(msg_hbm.at[pl.ds(off, tail)], mt_v)
            pltpu.sync_copy(mt_v, ssum.at[it_v], add=True)
        plsc.subcore_barrier()

        pltpu.sync_copy(ssum.at[pl.ds(sid * rpt, rpt)],
                        psum_hbm.at[cid, pl.ds(sid * rpt, rpt)])

    (psum,) = scatter_kernel(msg, idx1, sum_init)
    return psum


def _tc_stats(g1, g2, edge, wa, wb, we, b2d, tb, eoffb):
    e, d = g1.shape
    d2 = b2d.shape[1]
    de = edge.shape[1]
    grid = (e // tb,)

    def body(g1_ref, g2_ref, e_ref, wa_ref, wb_ref, we_ref, b_ref,
             out_ref, z_ref):
        i = pl.program_id(0)
        z = (
            jnp.dot(g1_ref[...].astype(jnp.bfloat16), wa_ref[...],
                    preferred_element_type=jnp.float32)
            + jnp.dot(g2_ref[...].astype(jnp.bfloat16), wb_ref[...],
                      preferred_element_type=jnp.float32)
            + jnp.dot(e_ref[...].astype(jnp.bfloat16), we_ref[...],
                      preferred_element_type=jnp.float32)
            + b_ref[...]
        )
        z_ref[...] = z.astype(jnp.bfloat16)

        @pl.when(i == 0)
        def _():
            out_ref[...] = jnp.zeros_like(out_ref)

        out_ref[0:1, :] += jnp.sum(z, axis=0, keepdims=True)
        out_ref[1:2, :] += jnp.sum(z * z, axis=0, keepdims=True)

    return pl.pallas_call(
        body,
        grid=grid,
        in_specs=[
            pl.BlockSpec((tb, d), lambda i: (i, 0)),
            pl.BlockSpec((tb, d), lambda i: (i, 0)),
            pl.BlockSpec((tb, de), lambda i: (i + eoffb, 0)),
            pl.BlockSpec((d, d2), lambda i: (0, 0)),
            pl.BlockSpec((d, d2), lambda i: (0, 0)),
            pl.BlockSpec((de, d2), lambda i: (0, 0)),
            pl.BlockSpec((1, d2), lambda i: (0, 0)),
        ],
        out_specs=[
            pl.BlockSpec((2, d2), lambda i: (0, 0)),
            pl.BlockSpec((tb, d2), lambda i: (i, 0)),
        ],
        out_shape=[
            jax.ShapeDtypeStruct((2, d2), jnp.float32),
            jax.ShapeDtypeStruct((e, d2), jnp.bfloat16),
        ],
    )(g1, g2, edge, wa, wb, we, b2d)


def _tc_msg(z, aff, d, tb):
    e, d2 = z.shape
    grid = (e // tb,)

    def body(z_ref, a_ref, out_ref):
        u = z_ref[...].astype(jnp.float32) * a_ref[0:1, :] + a_ref[1:2, :]
        gate = _sigmoid(u[:, :d])
        conv = _softplus(u[:, d:])
        out_ref[...] = gate * conv

    return pl.pallas_call(
        body,
        grid=grid,
        in_specs=[
            pl.BlockSpec((tb, d2), lambda i: (i, 0)),
            pl.BlockSpec((2, d2), lambda i: (0, 0)),
        ],
        out_specs=pl.BlockSpec((tb, d), lambda i: (i, 0)),
        out_shape=jax.ShapeDtypeStruct((e, d), jnp.float32),
    )(z, aff)


def _tc_final(node_fea, psum, pcnt, g2d, b2d, tn):
    n, d = node_fea.shape
    nt = n // tn
    nf = float(n)

    def body(node_ref, s_ref, c_ref, g_ref, b_ref, out_ref, acc1, acc2):
        p = pl.program_id(0)
        t = pl.program_id(1)
        s = s_ref[0] + s_ref[1]
        cnt = c_ref[0, :, 0:1] + c_ref[1, :, 0:1]
        agg = s / jnp.maximum(cnt, 1.0)

        @pl.when((p == 0) & (t == 0))
        def _():
            acc1[...] = jnp.zeros_like(acc1)
            acc2[...] = jnp.zeros_like(acc2)

        @pl.when(p == 0)
        def _():
            acc1[...] += jnp.sum(agg, axis=0, keepdims=True)
            acc2[...] += jnp.sum(agg * agg, axis=0, keepdims=True)
            out_ref[...] = agg

        @pl.when(p == 1)
        def _():
            mu = acc1[...] / nf
            var = acc2[...] / nf - mu * mu
            inv = 1.0 / jnp.sqrt(var + EPS)
            bn = (agg - mu) * inv * g_ref[...] + b_ref[...]
            out_ref[...] = _softplus(node_ref[...] + bn)

    return pl.pallas_call(
        body,
        grid=(2, nt),
        in_specs=[
            pl.BlockSpec((tn, d), lambda p, t: (t, 0)),
            pl.BlockSpec((NC, tn, d), lambda p, t: (0, t, 0)),
            pl.BlockSpec((NC, tn, d), lambda p, t: (0, t, 0)),
            pl.BlockSpec((1, d), lambda p, t: (0, 0)),
            pl.BlockSpec((1, d), lambda p, t: (0, 0)),
        ],
        out_specs=pl.BlockSpec((tn, d), lambda p, t: (t, 0)),
        out_shape=jax.ShapeDtypeStruct((n, d), jnp.float32),
        scratch_shapes=[
            pltpu.VMEM((1, d), jnp.float32),
            pltpu.VMEM((1, d), jnp.float32),
        ],
    )(node_fea, psum, pcnt, g2d, b2d)


def kernel(node_fea, idx1, idx2, edge_fea, W_full, b_full, gamma1, beta1,
           gamma2, beta2):
    n, d = node_fea.shape
    e = idx1.shape[0]
    d2 = 2 * d
    rpt = (-(-n // NS) + 7) // 8 * 8
    n_pad = NS * rpt

    wa = W_full[:, :d].T.astype(jnp.bfloat16)
    wb = W_full[:, d:2 * d].T.astype(jnp.bfloat16)
    we = W_full[:, 2 * d:].T.astype(jnp.bfloat16)
    b2d = b_full.reshape(1, d2)
    zpart = jnp.zeros((NC, n_pad, d), jnp.float32)

    h = e // 2
    g1a, g2a, pca = _sc_gather(node_fea, idx1, idx2, zpart, 0, h)
    g1b, g2b, pcnt = _sc_gather(node_fea, idx1, idx2, pca, h, h)

    tb = 1600
    s12a, za = _tc_stats(g1a, g2a, edge_fea, wa, wb, we, b2d, tb, 0)
    s12b, zb = _tc_stats(g1b, g2b, edge_fea, wa, wb, we, b2d, tb, h // tb)
    s12 = s12a + s12b
    mean = s12[0] / e
    var = s12[1] / e - mean * mean
    a = gamma1 / jnp.sqrt(var + EPS)
    c = beta1 - mean * a
    aff = jnp.stack([a, c])

    ma = _tc_msg(za, aff, d, tb=1600)
    mb = _tc_msg(zb, aff, d, tb=1600)

    psa = _sc_scatter(ma, idx1, n, zpart, 0)
    psum = _sc_scatter(mb, idx1, n, psa, h)

    out = _tc_final(node_fea, psum, pcnt, gamma2.reshape(1, d),
                    beta2.reshape(1, d), tn=2000)
    return out
